# baseline (speedup 1.0000x reference)
import functools

import jax
import jax.numpy as jnp
import numpy as np
from jax import lax
from jax.experimental import pallas as pl
from jax.experimental.pallas import tpu as pltpu
from jax.experimental.pallas import tpu_sc as plsc

N = 4096
D = 128
DE = 4
H = 2
DH = 64
C = 16

THR = np.float32(0.996)
NEG = np.float32(-1e30)

NC = 2
NS = 16
NW = NC * NS
ROWS_PER_W = N // NW


def _pre_body(x0_ref, win_ref, bin_ref, a4_ref, b4_ref, h_ref, sv_ref, hm_ref):
    h = jnp.dot(x0_ref[...], win_ref[...], preferred_element_type=jnp.float32)
    h = jnp.maximum(h + bin_ref[...], 0.0)
    h_ref[...] = h
    sv = jnp.dot(h, a4_ref[...], preferred_element_type=jnp.float32)
    sv_ref[...] = sv + b4_ref[...]
    hm = jnp.sum(h, axis=0, keepdims=True) * jnp.float32(1.0 / N)
    hm_ref[...] = jnp.broadcast_to(hm, (8, D))


def _tc_pre(x0, W_in, b_in, A4, B4):
    return pl.pallas_call(
        _pre_body,
        out_shape=(
            jax.ShapeDtypeStruct((N, D), jnp.float32),
            jax.ShapeDtypeStruct((N, 4), jnp.float32),
            jax.ShapeDtypeStruct((8, D), jnp.float32),
        ),
    )(x0, W_in, b_in, A4, B4)


def _post_body(oh_ref, wout_ref, bout_ref, out_ref):
    logits = jnp.dot(oh_ref[...], wout_ref[...], preferred_element_type=jnp.float32)
    logits = logits + bout_ref[...]
    m = jnp.max(logits, axis=1, keepdims=True)
    e = jnp.exp(logits - m)
    out_ref[...] = e / jnp.sum(e, axis=1, keepdims=True)


def _tc_post(out_h, W_out, b_out):
    return pl.pallas_call(
        _post_body,
        out_shape=jax.ShapeDtypeStruct((N, C), jnp.float32),
    )(out_h, W_out, b_out)


def _lane_bcast(v, k):
    lane = lax.iota(jnp.int32, 16)
    return jnp.sum(jnp.where(lane == k, v, 0.0))


def _sc_body(net_hbm, x1f_hbm, h_hbm, svf_hbm, hm_hbm, we_hbm, out_hbm,
             row_v, hits_v, idx16_v, idx64_v, x1g_v, hrows_v, sv_v, hmean_v,
             we_v, acc_v, sem1, sem2):
    wid = lax.axis_index("s") * NC + lax.axis_index("c")
    base_row = wid * ROWS_PER_W

    pltpu.sync_copy(svf_hbm, sv_v)
    pltpu.sync_copy(hm_hbm.at[0], hmean_v)
    pltpu.sync_copy(we_hbm, we_v)

    def zero_fn(i, _):
        hits_v[pl.ds(i * 16, 16)] = jnp.zeros((16,), jnp.int32)
        return 0
    lax.fori_loop(0, (N + 16) // 16, zero_fn, 0)

    lane = lax.iota(jnp.int32, 16)
    we = [plsc.load_gather(we_v, [jnp.full((16,), j, jnp.int32)])
          for j in range(8)]
    hm_regs = [hmean_v[pl.ds(j * 16, 16)] for j in range(8)]

    def row_fn(r, _):
        n = base_row + r
        pltpu.sync_copy(net_hbm.at[n], row_v)

        def scan_fn(c, cnt):
            v = row_v[pl.ds(c * 16, 16)]
            m = v > THR
            mi = m.astype(jnp.int32)
            pos = plsc.cumsum(mi) + (cnt - 1)
            plsc.store_scatter(hits_v, [pos], lane + c * 16, mask=m)
            return cnt + jnp.sum(mi)

        cnt = lax.fori_loop(0, N // 16, scan_fn, jnp.int32(0))
        nch = (cnt + 15) >> 4

        ssplat = jnp.full((16,), n * 4, jnp.int32)
        ss0 = plsc.load_gather(sv_v, [ssplat])
        ss1 = plsc.load_gather(sv_v, [ssplat + 1])

        def agg_fn(c, carry):
            m0, m1, s0, s1, acc = carry
            valid = (c * 16 + lane) < cnt
            idx = jnp.where(valid, hits_v[pl.ds(c * 16, 16)], 0)
            idx16_v[...] = idx
            i4 = n * (N * DE) + idx * DE
            for de in range(DE):
                idx64_v[pl.ds(de * 16, 16)] = i4 + de
            cp1 = pltpu.async_copy(x1f_hbm.at[idx64_v], x1g_v, sem1)
            cp2 = pltpu.async_copy(h_hbm.at[idx16_v], hrows_v, sem2)
            sd0 = plsc.load_gather(sv_v, [idx * 4 + 2])
            sd1 = plsc.load_gather(sv_v, [idx * 4 + 3])
            cp1.wait()
            xg = [x1g_v[pl.ds(de * 16, 16)] for de in range(DE)]
            e0 = xg[0] * we[0] + xg[1] * we[1] + xg[2] * we[2] + xg[3] * we[3]
            e1 = xg[0] * we[4] + xg[1] * we[5] + xg[2] * we[6] + xg[3] * we[7]
            sc0 = ss0 + sd0 + e0
            sc1 = ss1 + sd1 + e1
            sc0 = jnp.where(sc0 > 0, sc0, 0.2 * sc0)
            sc1 = jnp.where(sc1 > 0, sc1, 0.2 * sc1)
            sc0 = jnp.where(valid, sc0, NEG)
            sc1 = jnp.where(valid, sc1, NEG)
            mn0 = jnp.maximum(m0, jnp.max(sc0))
            mn1 = jnp.maximum(m1, jnp.max(sc1))
            w0 = jnp.exp(sc0 - mn0)
            w1 = jnp.exp(sc1 - mn1)
            scl0 = jnp.exp(jnp.full((16,), m0 - mn0, jnp.float32))
            scl1 = jnp.exp(jnp.full((16,), m1 - mn1, jnp.float32))
            ns0 = s0 * scl0 + w0
            ns1 = s1 * scl1 + w1
            cp2.wait()
            acc = [acc[j] * (scl0 if j < 4 else scl1) for j in range(8)]
            for k in range(16):
                w0k = jnp.full((16,), _lane_bcast(w0, k), jnp.float32)
                w1k = jnp.full((16,), _lane_bcast(w1, k), jnp.float32)
                for j in range(4):
                    acc[j] = acc[j] + w0k * hrows_v[k, pl.ds(j * 16, 16)]
                    acc[4 + j] = acc[4 + j] + w1k * hrows_v[k, pl.ds(64 + j * 16, 16)]
            return mn0, mn1, ns0, ns1, acc

        zero = jnp.zeros((16,), jnp.float32)
        init = (NEG, NEG, zero, zero, [zero] * 8)
        m0, m1, s0v, s1v, acc = lax.fori_loop(0, nch, agg_fn, init)

        has = (cnt > 0).astype(jnp.float32)
        hasv = jnp.full((16,), has, jnp.float32)
        s0 = jnp.full((16,), jnp.sum(s0v), jnp.float32)
        s1 = jnp.full((16,), jnp.sum(s1v), jnp.float32)
        inv0 = hasv / jnp.where(s0 > 0, s0, 1.0)
        inv1 = hasv / jnp.where(s1 > 0, s1, 1.0)
        hmw = 1.0 - hasv
        for j in range(8):
            res = acc[j] * (inv0 if j < 4 else inv1) + hmw * hm_regs[j]
            acc_v[pl.ds(j * 16, 16)] = res
        pltpu.sync_copy(acc_v, out_hbm.at[n])
        return 0

    lax.fori_loop(0, ROWS_PER_W, row_fn, 0)


def _sc_gat(network, x1f, h, svf, hm, we16):
    mesh = plsc.VectorSubcoreMesh(core_axis_name="c", subcore_axis_name="s")
    f = functools.partial(
        pl.kernel,
        out_type=jax.ShapeDtypeStruct((N, D), jnp.float32),
        mesh=mesh,
        compiler_params=pltpu.CompilerParams(needs_layout_passes=False),
        scratch_types=[
            pltpu.VMEM((N,), jnp.float32),
            pltpu.VMEM((N + 16,), jnp.int32),
            pltpu.VMEM((16,), jnp.int32),
            pltpu.VMEM((64,), jnp.int32),
            pltpu.VMEM((64,), jnp.float32),
            pltpu.VMEM((16, D), jnp.float32),
            pltpu.VMEM((4 * N,), jnp.float32),
            pltpu.VMEM((D,), jnp.float32),
            pltpu.VMEM((128,), jnp.float32),
            pltpu.VMEM((D,), jnp.float32),
            pltpu.SemaphoreType.DMA,
            pltpu.SemaphoreType.DMA,
        ],
    )(_sc_body)
    return f(network, x1f, h, svf, hm, we16)


def kernel(x0, x1, network, W_in, b_in, W_e, b_e, a_src, a_dst, W_out, b_out):
    f32 = jnp.float32
    A4 = jnp.zeros((D, 4), f32)
    A4 = A4.at[0:DH, 0].set(a_src[0])
    A4 = A4.at[DH:D, 1].set(a_src[1])
    A4 = A4.at[0:DH, 2].set(a_dst[0])
    A4 = A4.at[DH:D, 3].set(a_dst[1])
    B4 = jnp.concatenate([b_e, jnp.zeros((2,), f32)]).reshape(1, 4)

    h, sv, hm = _tc_pre(x0, W_in, b_in.reshape(1, D), A4, B4)

    we16 = jnp.concatenate([W_e.T.reshape(8), jnp.zeros((120,), f32)])
    x1f = x1.reshape(N * N * DE)
    svf = sv.reshape(4 * N)

    out_h = _sc_gat(network, x1f, h, svf, hm, we16)

    return _tc_post(out_h, W_out, b_out.reshape(1, C))


# speedup vs baseline: 18.9421x; 18.9421x over previous
import functools

import jax
import jax.numpy as jnp
import numpy as np
from jax import lax
from jax.experimental import pallas as pl
from jax.experimental.pallas import tpu as pltpu
from jax.experimental.pallas import tpu_sc as plsc

N = 4096
D = 128
DE = 4
H = 2
DH = 64
C = 16

THR = np.float32(0.996)
NEG = np.float32(-1e30)

NC = 2
NS = 16
NW = NC * NS
ROWS_PER_W = N // NW


def _pre_body(x0_ref, win_ref, bin_ref, a4_ref, b4_ref, h_ref, sv_ref, hm_ref):
    h = jnp.dot(x0_ref[...], win_ref[...], preferred_element_type=jnp.float32)
    h = jnp.maximum(h + bin_ref[...], 0.0)
    h_ref[...] = h
    sv = jnp.dot(h, a4_ref[...], preferred_element_type=jnp.float32)
    sv_ref[...] = sv + b4_ref[...]
    hm = jnp.sum(h, axis=0, keepdims=True) * jnp.float32(1.0 / N)
    hm_ref[...] = jnp.broadcast_to(hm, (8, D))


def _tc_pre(x0, W_in, b_in, A4, B4):
    return pl.pallas_call(
        _pre_body,
        out_shape=(
            jax.ShapeDtypeStruct((N, D), jnp.float32),
            jax.ShapeDtypeStruct((N, 4), jnp.float32),
            jax.ShapeDtypeStruct((8, D), jnp.float32),
        ),
    )(x0, W_in, b_in, A4, B4)


def _post_body(oh_ref, wout_ref, bout_ref, out_ref):
    logits = jnp.dot(oh_ref[...], wout_ref[...], preferred_element_type=jnp.float32)
    logits = logits + bout_ref[...]
    m = jnp.max(logits, axis=1, keepdims=True)
    e = jnp.exp(logits - m)
    out_ref[...] = e / jnp.sum(e, axis=1, keepdims=True)


def _tc_post(out_h, W_out, b_out):
    return pl.pallas_call(
        _post_body,
        out_shape=jax.ShapeDtypeStruct((N, C), jnp.float32),
    )(out_h, W_out, b_out)


def _lane_bcast(v, k):
    return jnp.full((16,), v[k], v.dtype)


def _sc_body(net_hbm, x1f_hbm, h_hbm, svf_hbm, hm_hbm, we_hbm, out_hbm,
             row_v, hits_v, idx16_v, idx64_v, x1g_v, hrows_v, sv_v, hmean_v,
             we_v, acc_v, sem1, sem2):
    wid = lax.axis_index("s") * NC + lax.axis_index("c")
    base_row = wid * ROWS_PER_W

    pltpu.sync_copy(svf_hbm, sv_v)
    pltpu.sync_copy(hm_hbm.at[0], hmean_v)
    pltpu.sync_copy(we_hbm, we_v)

    def zero_fn(i, _):
        hits_v[pl.ds(i * 16, 16)] = jnp.zeros((16,), jnp.int32)
        return 0
    lax.fori_loop(0, (N + 16) // 16, zero_fn, 0)

    lane = lax.iota(jnp.int32, 16)
    we = [plsc.load_gather(we_v, [jnp.full((16,), j, jnp.int32)])
          for j in range(8)]
    hm_regs = [hmean_v[pl.ds(j * 16, 16)] for j in range(8)]

    def row_fn(r, _):
        n = base_row + r
        pltpu.sync_copy(net_hbm.at[n], row_v)

        def scan_fn(c, cntv):
            v = row_v[pl.ds(c * 16, 16)]
            m = v > THR
            cs = plsc.cumsum(m.astype(jnp.int32))
            pos = cntv + cs - 1
            plsc.store_scatter(hits_v, [pos], lane + c * 16, mask=m)
            return cntv + _lane_bcast(cs, 15)

        cntv = lax.fori_loop(0, N // 16, scan_fn, jnp.zeros((16,), jnp.int32))
        cnt = cntv[0]
        nch = (cnt + 15) >> 4

        ssplat = jnp.full((16,), n * 4, jnp.int32)
        ss0 = plsc.load_gather(sv_v, [ssplat])
        ss1 = plsc.load_gather(sv_v, [ssplat + 1])

        def agg_fn(c, carry):
            m0, m1, s0, s1, acc = carry
            valid = (c * 16 + lane) < cnt
            idx = jnp.where(valid, hits_v[pl.ds(c * 16, 16)], 0)
            idx16_v[...] = idx
            i4 = n * (N * DE) + (idx >> 7) * (DE * 128) + (idx & 127)
            for de in range(DE):
                idx64_v[pl.ds(de * 16, 16)] = i4 + de * 128
            cp1 = pltpu.async_copy(x1f_hbm.at[idx64_v], x1g_v, sem1)
            cp2 = pltpu.async_copy(h_hbm.at[idx16_v], hrows_v, sem2)
            sd0 = plsc.load_gather(sv_v, [idx * 4 + 2])
            sd1 = plsc.load_gather(sv_v, [idx * 4 + 3])
            cp1.wait()
            xg = [x1g_v[pl.ds(de * 16, 16)] for de in range(DE)]
            e0 = xg[0] * we[0] + xg[1] * we[1] + xg[2] * we[2] + xg[3] * we[3]
            e1 = xg[0] * we[4] + xg[1] * we[5] + xg[2] * we[6] + xg[3] * we[7]
            sc0 = ss0 + sd0 + e0
            sc1 = ss1 + sd1 + e1
            sc0 = jnp.where(sc0 > 0, sc0, 0.2 * sc0)
            sc1 = jnp.where(sc1 > 0, sc1, 0.2 * sc1)
            sc0 = jnp.where(valid, sc0, NEG)
            sc1 = jnp.where(valid, sc1, NEG)
            mn0 = jnp.maximum(m0, jnp.max(sc0))
            mn1 = jnp.maximum(m1, jnp.max(sc1))
            w0 = jnp.exp(sc0 - mn0)
            w1 = jnp.exp(sc1 - mn1)
            scl0 = jnp.exp(jnp.full((16,), m0 - mn0, jnp.float32))
            scl1 = jnp.exp(jnp.full((16,), m1 - mn1, jnp.float32))
            ns0 = s0 * scl0 + w0
            ns1 = s1 * scl1 + w1
            cp2.wait()
            acc = [acc[j] * (scl0 if j < 4 else scl1) for j in range(8)]
            for k in range(16):
                w0k = jnp.full((16,), _lane_bcast(w0, k), jnp.float32)
                w1k = jnp.full((16,), _lane_bcast(w1, k), jnp.float32)
                for j in range(4):
                    acc[j] = acc[j] + w0k * hrows_v[k, pl.ds(j * 16, 16)]
                    acc[4 + j] = acc[4 + j] + w1k * hrows_v[k, pl.ds(64 + j * 16, 16)]
            return mn0, mn1, ns0, ns1, acc

        zero = jnp.zeros((16,), jnp.float32)
        init = (NEG, NEG, zero, zero, [zero] * 8)
        m0, m1, s0v, s1v, acc = lax.fori_loop(0, nch, agg_fn, init)

        has = (cnt > 0).astype(jnp.float32)
        hasv = jnp.full((16,), has, jnp.float32)
        s0 = jnp.full((16,), jnp.sum(s0v), jnp.float32)
        s1 = jnp.full((16,), jnp.sum(s1v), jnp.float32)
        inv0 = hasv / jnp.where(s0 > 0, s0, 1.0)
        inv1 = hasv / jnp.where(s1 > 0, s1, 1.0)
        hmw = 1.0 - hasv
        for j in range(8):
            res = acc[j] * (inv0 if j < 4 else inv1) + hmw * hm_regs[j]
            acc_v[pl.ds(j * 16, 16)] = res
        pltpu.sync_copy(acc_v, out_hbm.at[n])
        return 0

    lax.fori_loop(0, ROWS_PER_W, row_fn, 0)


def _sc_gat(network, x1f, h, svf, hm, we16):
    mesh = plsc.VectorSubcoreMesh(core_axis_name="c", subcore_axis_name="s")
    f = functools.partial(
        pl.kernel,
        out_type=jax.ShapeDtypeStruct((N, D), jnp.float32),
        mesh=mesh,
        compiler_params=pltpu.CompilerParams(needs_layout_passes=False),
        scratch_types=[
            pltpu.VMEM((N,), jnp.float32),
            pltpu.VMEM((N + 16,), jnp.int32),
            pltpu.VMEM((16,), jnp.int32),
            pltpu.VMEM((64,), jnp.int32),
            pltpu.VMEM((64,), jnp.float32),
            pltpu.VMEM((16, D), jnp.float32),
            pltpu.VMEM((4 * N,), jnp.float32),
            pltpu.VMEM((D,), jnp.float32),
            pltpu.VMEM((128,), jnp.float32),
            pltpu.VMEM((D,), jnp.float32),
            pltpu.SemaphoreType.DMA,
            pltpu.SemaphoreType.DMA,
        ],
    )(_sc_body)
    return f(network, x1f, h, svf, hm, we16)


def kernel(x0, x1, network, W_in, b_in, W_e, b_e, a_src, a_dst, W_out, b_out):
    f32 = jnp.float32
    A4 = jnp.zeros((D, 4), f32)
    A4 = A4.at[0:DH, 0].set(a_src[0])
    A4 = A4.at[DH:D, 1].set(a_src[1])
    A4 = A4.at[0:DH, 2].set(a_dst[0])
    A4 = A4.at[DH:D, 3].set(a_dst[1])
    B4 = jnp.concatenate([b_e, jnp.zeros((2,), f32)]).reshape(1, 4)

    h, sv, hm = _tc_pre(x0, W_in, b_in.reshape(1, D), A4, B4)

    we16 = jnp.concatenate([W_e.T.reshape(8), jnp.zeros((120,), f32)])
    x1f = x1.reshape(N, N // 128, 128, DE).transpose(0, 1, 3, 2).reshape(N * N * DE)
    svf = sv.reshape(4 * N)

    out_h = _sc_gat(network, x1f, h, svf, hm, we16)

    return _tc_post(out_h, W_out, b_out.reshape(1, C))


# speedup vs baseline: 19.1747x; 1.0123x over previous
import functools

import jax
import jax.numpy as jnp
import numpy as np
from jax import lax
from jax.experimental import pallas as pl
from jax.experimental.pallas import tpu as pltpu
from jax.experimental.pallas import tpu_sc as plsc

N = 4096
D = 128
DE = 4
H = 2
DH = 64
C = 16

THR = np.float32(0.996)
NEG = np.float32(-1e30)

NC = 2
NS = 16
NW = NC * NS
ROWS_PER_W = N // NW


def _pre_body(x0_ref, win_ref, bin_ref, a4_ref, b4_ref, h_ref, sv_ref, hm_ref):
    h = jnp.dot(x0_ref[...], win_ref[...], preferred_element_type=jnp.float32)
    h = jnp.maximum(h + bin_ref[...], 0.0)
    h_ref[...] = h
    sv = jnp.dot(h, a4_ref[...], preferred_element_type=jnp.float32)
    sv_ref[...] = sv + b4_ref[...]
    hm = jnp.sum(h, axis=0, keepdims=True) * jnp.float32(1.0 / N)
    hm_ref[...] = jnp.broadcast_to(hm, (8, D))


def _tc_pre(x0, W_in, b_in, A4, B4):
    return pl.pallas_call(
        _pre_body,
        out_shape=(
            jax.ShapeDtypeStruct((N, D), jnp.float32),
            jax.ShapeDtypeStruct((N, 4), jnp.float32),
            jax.ShapeDtypeStruct((8, D), jnp.float32),
        ),
    )(x0, W_in, b_in, A4, B4)


def _post_body(oh_ref, wout_ref, bout_ref, out_ref):
    logits = jnp.dot(oh_ref[...], wout_ref[...], preferred_element_type=jnp.float32)
    logits = logits + bout_ref[...]
    m = jnp.max(logits, axis=1, keepdims=True)
    e = jnp.exp(logits - m)
    out_ref[...] = e / jnp.sum(e, axis=1, keepdims=True)


def _tc_post(out_h, W_out, b_out):
    return pl.pallas_call(
        _post_body,
        out_shape=jax.ShapeDtypeStruct((N, C), jnp.float32),
    )(out_h, W_out, b_out)


def _lane_bcast(v, k):
    return jnp.full((16,), v[k], v.dtype)


def _sc_body(net_hbm, x1f_hbm, h_hbm, svf_hbm, hm_hbm, we_hbm, out_hbm,
             row_v, hits_v, idx16_v, idx64_v, x1g_v, hrows_v, sv_v, hmean_v,
             we_v, acc_v, sem1, sem2, semr):
    wid = lax.axis_index("s") * NC + lax.axis_index("c")
    base_row = wid * ROWS_PER_W

    pltpu.sync_copy(svf_hbm, sv_v)
    pltpu.sync_copy(hm_hbm.at[0], hmean_v)
    pltpu.sync_copy(we_hbm, we_v)

    def zero_fn(i, _):
        hits_v[pl.ds(i * 16, 16)] = jnp.zeros((16,), jnp.int32)
        return 0
    lax.fori_loop(0, (N + 16) // 16, zero_fn, 0)

    lane = lax.iota(jnp.int32, 16)
    we = [plsc.load_gather(we_v, [jnp.full((16,), j, jnp.int32)])
          for j in range(8)]
    hm_regs = [hmean_v[pl.ds(j * 16, 16)] for j in range(8)]

    pltpu.async_copy(net_hbm.at[base_row], row_v.at[pl.ds(0, N)], semr)

    def row_fn(r, _):
        n = base_row + r
        par = (r & 1) * N
        pltpu.make_async_copy(net_hbm.at[n], row_v.at[pl.ds(par, N)], semr).wait()
        nxt = jnp.minimum(n + 1, jnp.int32(N - 1))
        pltpu.async_copy(net_hbm.at[nxt], row_v.at[pl.ds(N - par, N)], semr)

        def scan_fn(g, cntv):
            o = par + g * 64
            vs = [row_v[pl.ds(o + i * 16, 16)] for i in range(4)]
            css = [plsc.cumsum((v > THR).astype(jnp.int32)) for v in vs]
            e = [cs[15] for cs in css]
            off = cntv - 1
            for i in range(4):
                plsc.store_scatter(hits_v, [off + css[i]], lane + (g * 64 + i * 16),
                                   mask=vs[i] > THR)
                off = off + e[i]
            return off + 1

        cntv = lax.fori_loop(0, N // 64, scan_fn, jnp.zeros((16,), jnp.int32))
        cnt = cntv[0]
        nch = (cnt + 15) >> 4

        ssplat = jnp.full((16,), n * 4, jnp.int32)
        ss0 = plsc.load_gather(sv_v, [ssplat])
        ss1 = plsc.load_gather(sv_v, [ssplat + 1])

        def agg_fn(c, carry):
            m0, m1, s0, s1, acc = carry
            valid = (c * 16 + lane) < cnt
            idx = jnp.where(valid, hits_v[pl.ds(c * 16, 16)], 0)
            idx = jnp.where(idx < N, idx, 0)
            idx16_v[...] = idx
            i4 = n * (N * DE) + (idx >> 7) * (DE * 128) + (idx & 127)
            for de in range(DE):
                idx64_v[pl.ds(de * 16, 16)] = i4 + de * 128
            cp1 = pltpu.async_copy(x1f_hbm.at[idx64_v], x1g_v, sem1)
            cp2 = pltpu.async_copy(h_hbm.at[idx16_v], hrows_v, sem2)
            sd0 = plsc.load_gather(sv_v, [idx * 4 + 2])
            sd1 = plsc.load_gather(sv_v, [idx * 4 + 3])
            cp1.wait()
            xg = [x1g_v[pl.ds(de * 16, 16)] for de in range(DE)]
            e0 = xg[0] * we[0] + xg[1] * we[1] + xg[2] * we[2] + xg[3] * we[3]
            e1 = xg[0] * we[4] + xg[1] * we[5] + xg[2] * we[6] + xg[3] * we[7]
            sc0 = ss0 + sd0 + e0
            sc1 = ss1 + sd1 + e1
            sc0 = jnp.where(sc0 > 0, sc0, 0.2 * sc0)
            sc1 = jnp.where(sc1 > 0, sc1, 0.2 * sc1)
            sc0 = jnp.where(valid, sc0, NEG)
            sc1 = jnp.where(valid, sc1, NEG)
            mn0 = jnp.maximum(m0, jnp.max(sc0))
            mn1 = jnp.maximum(m1, jnp.max(sc1))
            w0 = jnp.exp(sc0 - mn0)
            w1 = jnp.exp(sc1 - mn1)
            scl0 = jnp.exp(jnp.full((16,), m0 - mn0, jnp.float32))
            scl1 = jnp.exp(jnp.full((16,), m1 - mn1, jnp.float32))
            ns0 = s0 * scl0 + w0
            ns1 = s1 * scl1 + w1
            cp2.wait()
            acc = [acc[j] * (scl0 if j < 4 else scl1) for j in range(8)]
            for k in range(16):
                w0k = jnp.full((16,), _lane_bcast(w0, k), jnp.float32)
                w1k = jnp.full((16,), _lane_bcast(w1, k), jnp.float32)
                for j in range(4):
                    acc[j] = acc[j] + w0k * hrows_v[k, pl.ds(j * 16, 16)]
                    acc[4 + j] = acc[4 + j] + w1k * hrows_v[k, pl.ds(64 + j * 16, 16)]
            return mn0, mn1, ns0, ns1, acc

        zero = jnp.zeros((16,), jnp.float32)
        init = (NEG, NEG, zero, zero, [zero] * 8)
        m0, m1, s0v, s1v, acc = lax.fori_loop(0, nch, agg_fn, init)

        has = (cnt > 0).astype(jnp.float32)
        hasv = jnp.full((16,), has, jnp.float32)
        s0 = jnp.full((16,), jnp.sum(s0v), jnp.float32)
        s1 = jnp.full((16,), jnp.sum(s1v), jnp.float32)
        inv0 = hasv / jnp.where(s0 > 0, s0, 1.0)
        inv1 = hasv / jnp.where(s1 > 0, s1, 1.0)
        hmw = 1.0 - hasv
        for j in range(8):
            res = acc[j] * (inv0 if j < 4 else inv1) + hmw * hm_regs[j]
            acc_v[pl.ds(j * 16, 16)] = res
        pltpu.sync_copy(acc_v, out_hbm.at[n])
        return 0

    lax.fori_loop(0, ROWS_PER_W, row_fn, 0)
    pltpu.make_async_copy(net_hbm.at[0], row_v.at[pl.ds(0, N)], semr).wait()


def _sc_gat(network, x1f, h, svf, hm, we16):
    mesh = plsc.VectorSubcoreMesh(core_axis_name="c", subcore_axis_name="s")
    f = functools.partial(
        pl.kernel,
        out_type=jax.ShapeDtypeStruct((N, D), jnp.float32),
        mesh=mesh,
        compiler_params=pltpu.CompilerParams(needs_layout_passes=False),
        scratch_types=[
            pltpu.VMEM((2 * N,), jnp.float32),
            pltpu.VMEM((N + 16,), jnp.int32),
            pltpu.VMEM((16,), jnp.int32),
            pltpu.VMEM((64,), jnp.int32),
            pltpu.VMEM((64,), jnp.float32),
            pltpu.VMEM((16, D), jnp.float32),
            pltpu.VMEM((4 * N,), jnp.float32),
            pltpu.VMEM((D,), jnp.float32),
            pltpu.VMEM((128,), jnp.float32),
            pltpu.VMEM((D,), jnp.float32),
            pltpu.SemaphoreType.DMA,
            pltpu.SemaphoreType.DMA,
            pltpu.SemaphoreType.DMA,
        ],
    )(_sc_body)
    return f(network, x1f, h, svf, hm, we16)


def kernel(x0, x1, network, W_in, b_in, W_e, b_e, a_src, a_dst, W_out, b_out):
    f32 = jnp.float32
    A4 = jnp.zeros((D, 4), f32)
    A4 = A4.at[0:DH, 0].set(a_src[0])
    A4 = A4.at[DH:D, 1].set(a_src[1])
    A4 = A4.at[0:DH, 2].set(a_dst[0])
    A4 = A4.at[DH:D, 3].set(a_dst[1])
    B4 = jnp.concatenate([b_e, jnp.zeros((2,), f32)]).reshape(1, 4)

    h, sv, hm = _tc_pre(x0, W_in, b_in.reshape(1, D), A4, B4)

    we16 = jnp.concatenate([W_e.T.reshape(8), jnp.zeros((120,), f32)])
    x1f = x1.reshape(N, N // 128, 128, DE).transpose(0, 1, 3, 2).reshape(N * N * DE)
    svf = sv.reshape(4 * N)

    out_h = _sc_gat(network, x1f, h, svf, hm, we16)

    return _tc_post(out_h, W_out, b_out.reshape(1, C))
